# baseline (speedup 1.0000x reference)
import jax
import jax.numpy as jnp
from jax.experimental import pallas as pl
from jax.experimental.pallas import tpu as pltpu

P = 0.2


def _drop_kernel(r_ref, xs_ref, zeros_ref, x_ref, o_ref, sem_big, sem_z):
    B, C, H, W = x_ref.shape

    def issue(b, _):
        pltpu.make_async_copy(x_ref.at[b], o_ref.at[b], sem_big.at[b]).start()
        return 0

    jax.lax.fori_loop(0, B, issue, 0)

    def finish(b, _):
        pltpu.make_async_copy(x_ref.at[b], o_ref.at[b], sem_big.at[b]).wait()
        r0 = r_ref[b, 0]
        r1 = r_ref[b, 1]

        def step(_, lohi):
            lo, hi = lohi
            mid = (lo + hi) // 2
            pred = r0 > xs_ref[0, mid]
            return jnp.where(pred, mid + 1, lo), jnp.where(pred, hi, mid)

        lo, _ = jax.lax.fori_loop(0, 8, step, (jnp.int32(0), jnp.int32(C)))

        @pl.when(r1 < P)
        def _():
            cp = pltpu.make_async_copy(
                zeros_ref.at[0, 0], o_ref.at[b, lo], sem_z.at[b]
            )
            cp.start()
            cp.wait()

        return 0

    jax.lax.fori_loop(0, B, finish, 0)


def kernel(tensor, r):
    B, C, H, W = tensor.shape
    xs = jnp.linspace(1.0 / C, 1.0, C).reshape(1, C).astype(jnp.float32)
    zeros = jnp.zeros((1, 1, H, W), jnp.float32)

    out = pl.pallas_call(
        _drop_kernel,
        in_specs=[
            pl.BlockSpec(memory_space=pltpu.SMEM),
            pl.BlockSpec(memory_space=pltpu.SMEM),
            pl.BlockSpec(memory_space=pltpu.MemorySpace.HBM),
            pl.BlockSpec(memory_space=pltpu.MemorySpace.HBM),
        ],
        out_specs=pl.BlockSpec(memory_space=pltpu.MemorySpace.HBM),
        out_shape=jax.ShapeDtypeStruct((B, C, H, W), jnp.float32),
        scratch_shapes=[
            pltpu.SemaphoreType.DMA((B,)),
            pltpu.SemaphoreType.DMA((B,)),
        ],
    )(r, xs, zeros, tensor)
    return out


# speedup vs baseline: 42.3041x; 42.3041x over previous
import dataclasses

import jax
import jax.numpy as jnp
from jax import lax
from jax.experimental import pallas as pl
from jax.experimental.pallas import tpu as pltpu
from jax.experimental.pallas import tpu_sc as plsc

P = 0.2


def _sc_body(rT_hbm, xs_hbm, z_hbm, x_hbm, o_hbm,
             r_v, xs_v, buf, sem_small, sem_in, sem_out):
    B, C, H, W = x_hbm.shape
    HB = H // 4
    NBLK = 48 * 4

    wid = lax.axis_index("s") * 2 + lax.axis_index("c")
    b = wid // 2
    c0 = (wid % 2) * 48

    pltpu.make_async_copy(rT_hbm, r_v, sem_small).start()
    pltpu.make_async_copy(rT_hbm, r_v, sem_small).wait()
    pltpu.make_async_copy(xs_hbm, xs_v, sem_small).start()
    pltpu.make_async_copy(xs_hbm, xs_v, sem_small).wait()
    lane = lax.iota(jnp.int32, 16)
    sel = (lane == b).astype(jnp.float32)
    r0b = jnp.sum(r_v[0, :] * sel, axis=0)
    r1b = jnp.sum(r_v[1, :] * sel, axis=0)
    cnt = jnp.int32(0)
    for k in range(C // 16):
        cnt = cnt + jnp.sum(
            (r0b > xs_v[pl.ds(k * 16, 16)]).astype(jnp.int32), axis=0)
    drop_mine = jnp.logical_and(
        r1b < P, jnp.logical_and(cnt >= c0, cnt < c0 + 48))

    def start_in(i, j):
        c = c0 + i // 4
        h0 = (i % 4) * HB
        pltpu.make_async_copy(
            x_hbm.at[b, c, pl.ds(h0, HB), :], buf.at[j], sem_in.at[j]
        ).start()

    def wait_in(j):
        pltpu.make_async_copy(
            x_hbm.at[0, 0, pl.ds(0, HB), :], buf.at[j], sem_in.at[j]
        ).wait()

    def start_out(i, j):
        c = c0 + i // 4
        h0 = (i % 4) * HB
        pltpu.make_async_copy(
            buf.at[j], o_hbm.at[b, c, pl.ds(h0, HB), :], sem_out.at[j]
        ).start()

    def wait_out(i, j):
        c = c0 + i // 4
        h0 = (i % 4) * HB
        pltpu.make_async_copy(
            buf.at[j], o_hbm.at[b, c, pl.ds(h0, HB), :], sem_out.at[j]
        ).wait()

    start_in(0, 0)
    start_in(1, 1)
    wait_in(0)
    start_out(0, 0)
    start_in(2, 2)
    wait_in(1)
    start_out(1, 1)

    @pl.loop(3, NBLK, step=3)
    def _(g0):
        for j in range(3):
            i = g0 + j
            wait_out(i - 3, j)
            start_in(i, j)
            jp = (j + 2) % 3
            wait_in(jp)
            start_out(i - 1, jp)

    wait_in(2)
    start_out(NBLK - 1, 2)
    wait_out(NBLK - 3, 0)
    wait_out(NBLK - 2, 1)
    wait_out(NBLK - 1, 2)

    @pl.when(drop_mine)
    def _():
        zb = buf.at[0]
        pltpu.make_async_copy(z_hbm, zb, sem_small).start()
        pltpu.make_async_copy(z_hbm, zb, sem_small).wait()
        for hb in range(4):
            pltpu.make_async_copy(
                zb, o_hbm.at[b, cnt, pl.ds(hb * HB, HB), :],
                sem_out.at[hb % 3],
            ).start()
        for hb in range(4):
            pltpu.make_async_copy(
                zb, o_hbm.at[b, cnt, pl.ds(hb * HB, HB), :],
                sem_out.at[hb % 3],
            ).wait()


def kernel(tensor, r):
    B, C, H, W = tensor.shape
    HB = H // 4
    xs = jnp.linspace(1.0 / C, 1.0, C).astype(jnp.float32)
    rT = r.T.astype(jnp.float32)
    zeros = jnp.zeros((HB, W), jnp.float32)

    mesh = plsc.VectorSubcoreMesh(core_axis_name="c", subcore_axis_name="s")
    cp = pltpu.CompilerParams()
    if "needs_layout_passes" in pltpu.CompilerParams.__dataclass_fields__:
        cp = dataclasses.replace(cp, needs_layout_passes=False)
    run = pl.kernel(
        _sc_body,
        out_type=jax.ShapeDtypeStruct((B, C, H, W), jnp.float32),
        mesh=mesh,
        compiler_params=cp,
        scratch_types=[
            pltpu.VMEM((2, 16), jnp.float32),
            pltpu.VMEM((96,), jnp.float32),
            pltpu.VMEM((3, HB, W), jnp.float32),
            pltpu.SemaphoreType.DMA,
            pltpu.SemaphoreType.DMA((3,)),
            pltpu.SemaphoreType.DMA((3,)),
        ],
    )
    return run(rT, xs, zeros, tensor)
